# baseline (speedup 1.0000x reference)
import functools

import jax
import jax.numpy as jnp
from jax import lax
from jax.experimental import pallas as pl
from jax.experimental.pallas import tpu as pltpu
from jax.experimental.pallas import tpu_sc as plsc

N_EXP = 64
TOPK = 8
ROW_BLOCK = 1024
N_TOK = 16384
C_PER = N_TOK // 32


def _keys_block(x_ref, w_ref, keys_ref):
    _IDX_MASK = jnp.int32(N_EXP - 1)
    _SCALE = jnp.float32(1 << 21)
    x = x_ref[0]
    w = w_ref[...]
    logits_t = jax.lax.dot_general(
        w, x, (((1,), (1,)), ((), ())), preferred_element_type=jnp.float32
    )
    lq = (jnp.clip(logits_t, -8.0, 8.0) * _SCALE).astype(jnp.int32)
    iota = jax.lax.broadcasted_iota(jnp.int32, lq.shape, 0)
    keys_ref[...] = lq * N_EXP + (_IDX_MASK - iota)


_sc_mesh = plsc.VectorSubcoreMesh(core_axis_name="c", subcore_axis_name="s")


@functools.partial(
    pl.kernel,
    out_type=[
        jax.ShapeDtypeStruct((N_TOK * TOPK,), jnp.int32),
        jax.ShapeDtypeStruct((N_TOK * TOPK,), jnp.float32),
    ],
    mesh=_sc_mesh,
    scratch_types=[
        pltpu.VMEM((N_EXP * C_PER,), jnp.int32),
        pltpu.VMEM((C_PER * TOPK,), jnp.int32),
        pltpu.VMEM((C_PER * TOPK,), jnp.float32),
        pltpu.SemaphoreType.DMA,
    ],
)
def _sc_topk(keys_hbm, idx_hbm, wgt_hbm, keys_v, idx_v, wgt_v, sem):
    wid = lax.axis_index("s") * 2 + lax.axis_index("c")
    base = wid * C_PER
    copies = [
        pltpu.async_copy(
            keys_hbm.at[pl.ds(e * N_TOK + base, C_PER)],
            keys_v.at[pl.ds(e * C_PER, C_PER)],
            sem,
        )
        for e in range(N_EXP)
    ]
    for c in copies:
        c.wait()
    _NEG = jnp.int32(-(2**31) + 1)
    _I63 = jnp.int32(63)
    _VAL = jnp.int32(-64)
    _INV = jnp.float32(1.0 / (1 << 27))

    def group(g, carry):
        col = g * 16
        t = [jnp.full((16,), _NEG, jnp.int32) for _ in range(TOPK)]
        for e in range(N_EXP):
            v = keys_v[pl.ds(e * C_PER + col, 16)]
            for j in range(TOPK):
                hi = jnp.maximum(t[j], v)
                v = jnp.minimum(t[j], v)
                t[j] = hi
        l0 = (t[0] & _VAL).astype(jnp.float32) * _INV
        exs = []
        for j in range(TOPK):
            lj = (t[j] & _VAL).astype(jnp.float32) * _INV
            exs.append(jnp.exp(lj - l0))
        denom = exs[0]
        for j in range(1, TOPK):
            denom = denom + exs[j]
        for j in range(TOPK):
            idx_v[pl.ds(col * TOPK + j * 16, 16)] = _I63 - (t[j] & _I63)
            wgt_v[pl.ds(col * TOPK + j * 16, 16)] = exs[j] / denom
        return carry

    lax.fori_loop(0, C_PER // 16, group, 0)
    pltpu.sync_copy(idx_v, idx_hbm.at[pl.ds(base * TOPK, C_PER * TOPK)])
    pltpu.sync_copy(wgt_v, wgt_hbm.at[pl.ds(base * TOPK, C_PER * TOPK)])


@jax.jit
def kernel(hidden_states, weight, expert_biases):
    del expert_biases
    bsz, seq, d = hidden_states.shape
    n = bsz * seq
    blocks_per_batch = seq // ROW_BLOCK
    grid = (bsz, blocks_per_batch)
    keys = pl.pallas_call(
        _keys_block,
        grid=grid,
        in_specs=[
            pl.BlockSpec((1, ROW_BLOCK, d), lambda b, i: (b, i, 0)),
            pl.BlockSpec((N_EXP, d), lambda b, i: (0, 0)),
        ],
        out_specs=pl.BlockSpec(
            (N_EXP, ROW_BLOCK),
            lambda b, i, _nb=blocks_per_batch: (0, b * _nb + i),
        ),
        out_shape=jax.ShapeDtypeStruct((N_EXP, n), jnp.int32),
        compiler_params=pltpu.CompilerParams(
            dimension_semantics=("arbitrary", "arbitrary"),
        ),
    )(hidden_states, weight.astype(jnp.float32))
    idx_flat, wgt_flat = _sc_topk(keys.reshape(-1))
    idx = idx_flat.reshape(n // 16, TOPK, 16).transpose(0, 2, 1).reshape(n, TOPK)
    wgt = wgt_flat.reshape(n // 16, TOPK, 16).transpose(0, 2, 1).reshape(n, TOPK)
    return idx, wgt.astype(hidden_states.dtype)
